# Initial kernel scaffold; baseline (speedup 1.0000x reference)
import functools

import jax
import jax.numpy as jnp
from jax import lax
from jax.experimental import pallas as pl
from jax.experimental.pallas import tpu as pltpu
from jax.experimental.pallas import tpu_sc as plsc

_N = 10000
_D = 128
_E = 320000
_RB = 400
_CH = 128
_NSUB = 16
_ROWS_PER_TILE = _N // _NSUB
_ZR = 125
_NCHUNKS = _E // _CH
_Q, _REM = divmod(_NCHUNKS, _NSUB)


def _tc_body(mean_ref, var_ref, wm_ref, bm_ref, wv_ref, bv_ref, out_ref):
    dn = (((1,), (1,)), ((), ()))
    ml = lax.dot_general(mean_ref[...], wm_ref[...], dn,
                         preferred_element_type=jnp.float32) + bm_ref[...]
    vl = lax.dot_general(var_ref[...], wv_ref[...], dn,
                         preferred_element_type=jnp.float32) + bv_ref[...]
    me = jnp.where(ml > 0, ml, jnp.expm1(ml))
    vr = jnp.maximum(vl, 0.0)
    att = jnp.exp(-vr)
    out_ref[0] = me * att
    out_ref[1] = vr * att * att


def _tc_transform(mean, var, w_mean, b_mean, w_var, b_var):
    nb = _N // _RB
    return pl.pallas_call(
        _tc_body,
        grid=(nb,),
        in_specs=[
            pl.BlockSpec((_RB, _D), lambda b: (b, 0)),
            pl.BlockSpec((_RB, _D), lambda b: (b, 0)),
            pl.BlockSpec((_D, _D), lambda b: (0, 0)),
            pl.BlockSpec((1, _D), lambda b: (0, 0)),
            pl.BlockSpec((_D, _D), lambda b: (0, 0)),
            pl.BlockSpec((1, _D), lambda b: (0, 0)),
        ],
        out_specs=pl.BlockSpec((2, _RB, _D), lambda b: (0, b, 0)),
        out_shape=jax.ShapeDtypeStruct((2, _N, _D), jnp.float32),
    )(mean, var, w_mean, b_mean.reshape(1, _D), w_var, b_var.reshape(1, _D))


def _sc_body(x_hbm, cols_hbm, rows_hbm, adj_hbm, m_out, v_out,
             col_v, row_v, w_v, rows_v, zbuf, acc, sem):
    cid = lax.axis_index("c")
    sid = lax.axis_index("s")

    def _zrow(i, carry):
        for b in range(8):
            zbuf[i, pl.ds(b * 16, 16)] = jnp.zeros((16,), jnp.float32)
        return carry
    lax.fori_loop(0, _ZR, _zrow, 0)
    r0 = sid * _ROWS_PER_TILE
    for k in range(_ROWS_PER_TILE // _ZR):
        pltpu.sync_copy(zbuf, acc.at[pl.ds(r0 + k * _ZR, _ZR)])
    plsc.subcore_barrier()

    nch = jnp.where(sid < _REM, _Q + 1, _Q)
    start = sid * _Q + jnp.minimum(sid, _REM)

    def _chunk(j, carry):
        base = (start + j) * _CH
        pltpu.sync_copy(cols_hbm.at[cid, pl.ds(base, _CH)], col_v)
        pltpu.sync_copy(rows_hbm.at[pl.ds(base, _CH)], row_v)
        pltpu.sync_copy(adj_hbm.at[cid, pl.ds(base, _CH)], w_v)
        pltpu.async_copy(x_hbm.at[col_v], rows_v, sem).wait()

        def _scale(e, c2):
            wbc = plsc.load_gather(w_v, [jnp.full((16,), e, jnp.int32)])
            for b in range(8):
                sl = pl.ds(b * 16, 16)
                rows_v[e, sl] = rows_v[e, sl] * wbc
            return c2
        lax.fori_loop(0, _CH, _scale, 0)
        pltpu.sync_copy(rows_v, acc.at[row_v], add=True)
        return carry
    lax.fori_loop(0, nch, _chunk, 0)
    plsc.subcore_barrier()

    @pl.when(cid == 0)
    def _():
        pltpu.sync_copy(acc.at[pl.ds(r0, _ROWS_PER_TILE)],
                        m_out.at[pl.ds(r0, _ROWS_PER_TILE)])

    @pl.when(cid == 1)
    def _():
        pltpu.sync_copy(acc.at[pl.ds(r0, _ROWS_PER_TILE)],
                        v_out.at[pl.ds(r0, _ROWS_PER_TILE)])


_sc_aggregate = functools.partial(
    pl.kernel,
    out_type=[jax.ShapeDtypeStruct((_N, _D), jnp.float32),
              jax.ShapeDtypeStruct((_N, _D), jnp.float32)],
    mesh=plsc.VectorSubcoreMesh(core_axis_name="c", subcore_axis_name="s"),
    scratch_types=[
        pltpu.VMEM((_CH,), jnp.int32),
        pltpu.VMEM((_CH,), jnp.int32),
        pltpu.VMEM((_CH,), jnp.float32),
        pltpu.VMEM((_CH, _D), jnp.float32),
        pltpu.VMEM((_ZR, _D), jnp.float32),
        pltpu.VMEM_SHARED((_N, _D), jnp.float32),
        pltpu.SemaphoreType.DMA,
    ],
)(_sc_body)


def kernel(mean, var, edge_index, adj0_values, adj1_values,
           W_mean, b_mean, W_var, b_var):
    x_all = _tc_transform(mean, var, W_mean, b_mean, W_var, b_var)
    x_all = x_all.reshape(2 * _N, _D)
    col = edge_index[1]
    cols_all = jnp.stack([col, col + _N])
    adj_all = jnp.stack([adj0_values, adj1_values])
    m_out, v_out = _sc_aggregate(x_all, cols_all, edge_index[0], adj_all)
    return (m_out, v_out)


# speedup vs baseline: 4.8785x; 4.8785x over previous
import functools

import jax
import jax.numpy as jnp
from jax import lax
from jax.experimental import pallas as pl
from jax.experimental.pallas import tpu as pltpu
from jax.experimental.pallas import tpu_sc as plsc

_N = 10000
_D = 128
_E = 320000
_RB = 400
_CH = 128
_NSUB = 16
_RPT = 624
_ZR = 208
_NCHUNKS = _E // _CH
_Q, _REM = divmod(_NCHUNKS, _NSUB)


def _tc_body(mean_ref, var_ref, wm_ref, bm_ref, wv_ref, bv_ref, out_ref):
    dn = (((1,), (1,)), ((), ()))
    ml = lax.dot_general(mean_ref[...], wm_ref[...], dn,
                         preferred_element_type=jnp.float32) + bm_ref[...]
    vl = lax.dot_general(var_ref[...], wv_ref[...], dn,
                         preferred_element_type=jnp.float32) + bv_ref[...]
    me = jnp.where(ml > 0, ml, jnp.exp(ml) - 1.0)
    vr = jnp.maximum(vl, 0.0)
    att = jnp.exp(-vr)
    out_ref[0] = me * att
    out_ref[1] = vr * att * att


def _tc_transform(mean, var, w_mean, b_mean, w_var, b_var):
    nb = _N // _RB
    return pl.pallas_call(
        _tc_body,
        grid=(nb,),
        in_specs=[
            pl.BlockSpec((_RB, _D), lambda b: (b, 0)),
            pl.BlockSpec((_RB, _D), lambda b: (b, 0)),
            pl.BlockSpec((_D, _D), lambda b: (0, 0)),
            pl.BlockSpec((1, _D), lambda b: (0, 0)),
            pl.BlockSpec((_D, _D), lambda b: (0, 0)),
            pl.BlockSpec((1, _D), lambda b: (0, 0)),
        ],
        out_specs=pl.BlockSpec((2, _RB, _D), lambda b: (0, b, 0)),
        out_shape=jax.ShapeDtypeStruct((2, _N, _D), jnp.float32),
    )(mean, var, w_mean, b_mean.reshape(1, _D), w_var, b_var.reshape(1, _D))


def _sc_body(x_hbm, cols_hbm, rows_hbm, adj_hbm, m_out, v_out,
             col_v, row_v, w_v, rows_v, zbuf, acc, sem):
    cid = lax.axis_index("c")
    sid = lax.axis_index("s")

    def _zrow(i, carry):
        for b in range(8):
            zbuf[i, pl.ds(b * 16, 16)] = jnp.zeros((16,), jnp.float32)
        return carry
    lax.fori_loop(0, _ZR, _zrow, 0)
    r0 = sid * _RPT
    for k in range(_RPT // _ZR):
        pltpu.sync_copy(zbuf, acc.at[pl.ds(r0 + k * _ZR, _ZR)])

    @pl.when(sid == _NSUB - 1)
    def _():
        pltpu.sync_copy(zbuf.at[pl.ds(0, _N - _NSUB * _RPT)],
                        acc.at[pl.ds(_NSUB * _RPT, _N - _NSUB * _RPT)])
    plsc.subcore_barrier()

    nch = jnp.where(sid < _REM, _Q + 1, _Q)
    start = sid * _Q + jnp.minimum(sid, _REM)

    def _chunk(j, carry):
        base = (start + j) * _CH
        pltpu.sync_copy(cols_hbm.at[cid, pl.ds(base, _CH)], col_v)
        pltpu.sync_copy(rows_hbm.at[pl.ds(base, _CH)], row_v)
        pltpu.sync_copy(adj_hbm.at[cid, pl.ds(base, _CH)], w_v)
        pltpu.async_copy(x_hbm.at[col_v], rows_v, sem).wait()

        def _scale(g, c2):
            wv = w_v[pl.ds(g * 16, 16)]
            for k in range(16):
                wbc = jnp.broadcast_to(wv[k], (16,))
                e = g * 16 + k
                for b in range(8):
                    sl = pl.ds(b * 16, 16)
                    rows_v[e, sl] = rows_v[e, sl] * wbc
            return c2
        lax.fori_loop(0, _CH // 16, _scale, 0)
        pltpu.sync_copy(rows_v, acc.at[row_v], add=True)
        return carry
    lax.fori_loop(0, nch, _chunk, 0)
    plsc.subcore_barrier()

    _tail0 = _NSUB * _RPT
    _tailn = _N - _tail0

    @pl.when(cid == 0)
    def _():
        pltpu.sync_copy(acc.at[pl.ds(r0, _RPT)], m_out.at[pl.ds(r0, _RPT)])

        @pl.when(sid == _NSUB - 1)
        def _():
            pltpu.sync_copy(acc.at[pl.ds(_tail0, _tailn)],
                            m_out.at[pl.ds(_tail0, _tailn)])

    @pl.when(cid == 1)
    def _():
        pltpu.sync_copy(acc.at[pl.ds(r0, _RPT)], v_out.at[pl.ds(r0, _RPT)])

        @pl.when(sid == _NSUB - 1)
        def _():
            pltpu.sync_copy(acc.at[pl.ds(_tail0, _tailn)],
                            v_out.at[pl.ds(_tail0, _tailn)])


@functools.cache
def _sc_aggregate():
    return functools.partial(
        pl.kernel,
        out_type=[jax.ShapeDtypeStruct((_N, _D), jnp.float32),
                  jax.ShapeDtypeStruct((_N, _D), jnp.float32)],
        mesh=plsc.VectorSubcoreMesh(core_axis_name="c", subcore_axis_name="s",
                                    num_cores=2, num_subcores=_NSUB),
        scratch_types=[
            pltpu.VMEM((_CH,), jnp.int32),
            pltpu.VMEM((_CH,), jnp.int32),
            pltpu.VMEM((_CH,), jnp.float32),
            pltpu.VMEM((_CH, _D), jnp.float32),
            pltpu.VMEM((_ZR, _D), jnp.float32),
            pltpu.VMEM_SHARED((_N, _D), jnp.float32),
            pltpu.SemaphoreType.DMA,
        ],
    )(_sc_body)


def kernel(mean, var, edge_index, adj0_values, adj1_values,
           W_mean, b_mean, W_var, b_var):
    x_all = _tc_transform(mean, var, W_mean, b_mean, W_var, b_var)
    x_all = x_all.reshape(2 * _N, _D)
    col = edge_index[1]
    cols_all = jnp.stack([col, col + _N])
    adj_all = jnp.stack([adj0_values, adj1_values])
    m_out, v_out = _sc_aggregate()(x_all, cols_all, edge_index[0], adj_all)
    return (m_out, v_out)
